# baseline (speedup 1.0000x reference)
import functools

import jax
import jax.numpy as jnp
from jax import lax
from jax.experimental import pallas as pl
from jax.experimental.pallas import tpu as pltpu
from jax.experimental.pallas import tpu_sc as plsc

N = 10000
E = 320000
D = 128
DQ = D // 4

NC = 2
NS = 16
L = 16

EPAD = 327680
PER_T = EPAD // NS
ROWS_T = PER_T // 128
MC = 256
NCH = PER_T // MC
NR = NCH // 4
K = MC // 128
WB = 4 * MC
TROWS = NS * 640
ZR = TROWS // NS
SR = N // NS

_mesh = plsc.VectorSubcoreMesh(
    core_axis_name="c", subcore_axis_name="s", num_cores=NC, num_subcores=NS
)


@functools.partial(
    pl.kernel,
    out_type=jax.ShapeDtypeStruct((4, TROWS, DQ), jnp.float32),
    mesh=_mesh,
    scratch_types=[
        pltpu.VMEM((ROWS_T, 128), jnp.int32),
        pltpu.VMEM((ROWS_T, 128), jnp.int32),
        pltpu.VMEM((WB,), jnp.float32),
        pltpu.VMEM((MC, DQ), jnp.float32),
        pltpu.VMEM((MC, DQ), jnp.float32),
        pltpu.VMEM((MC, DQ), jnp.float32),
        pltpu.VMEM((MC, DQ), jnp.float32),
        pltpu.VMEM((128, DQ), jnp.float32),
        pltpu.VMEM_SHARED((TROWS, DQ), jnp.float32),
        pltpu.VMEM_SHARED((TROWS, DQ), jnp.float32),
        pltpu.SemaphoreType.DMA,
        pltpu.SemaphoreType.DMA,
        pltpu.SemaphoreType.DMA,
        pltpu.SemaphoreType.DMA,
        pltpu.SemaphoreType.DMA,
        pltpu.SemaphoreType.DMA,
        pltpu.SemaphoreType.DMA,
        pltpu.SemaphoreType.DMA,
        pltpu.SemaphoreType.DMA,
    ],
    compiler_params=pltpu.CompilerParams(use_tc_tiling_on_sc=False),
)
def _sc_aggregate(feat_hbm, src_hbm, dst_hbm, w_hbm, out_hbm,
                  src_v, dst_v, w_v, buf0, buf1, buf2, buf3, zero_v,
                  table, acc, sg0, sg1, sg2, sg3, ss0, ss1, ss2, ss3, sem_t):
    cid = lax.axis_index("c")
    sid = lax.axis_index("s")
    bufs = (buf0, buf1, buf2, buf3)
    gsems = (sg0, sg1, sg2, sg3)
    ssems = (ss0, ss1, ss2, ss3)

    erow = pl.multiple_of(sid * ROWS_T, 8)
    ecps = [
        pltpu.async_copy(src_hbm.at[pl.ds(erow, ROWS_T)], src_v, sem_t),
        pltpu.async_copy(dst_hbm.at[pl.ds(erow, ROWS_T)], dst_v, sem_t),
    ]

    @plsc.parallel_loop(0, 128, 1, unroll=2)
    def _zero_row(i):
        for j in range(DQ // L):
            zero_v[i, pl.ds(L * j, L)] = jnp.zeros((L,), jnp.float32)

    for cp in ecps:
        cp.wait()

    def _scale_buf(rows, woff):
        @plsc.parallel_loop(0, MC // L, 1, unroll=4)
        def _scale_16rows(t):
            w16 = w_v[pl.ds(woff + t * L, L)]
            for r in range(L):
                i = t * L + r
                wv = jnp.full((L,), w16[r], jnp.float32)
                for j in range(DQ // L):
                    rows[i, pl.ds(L * j, L)] = rows[i, pl.ds(L * j, L)] * wv

    def _fire_gather(c, b):
        for j in range(K):
            pltpu.async_copy(
                table.at[src_v.at[c * K + j]],
                bufs[b].at[pl.ds(j * 128, 128)],
                gsems[b],
            )

    def _drain_gather(b):
        for j in range(K):
            pltpu.make_async_copy(
                table.at[src_v.at[j]], bufs[b].at[pl.ds(j * 128, 128)], gsems[b]
            ).wait()

    def _fire_scatter(c, b):
        for j in range(K):
            pltpu.async_copy(
                bufs[b].at[pl.ds(j * 128, 128)],
                acc.at[dst_v.at[c * K + j]],
                ssems[b],
                add=True,
            )

    def _drain_scatter(b):
        for j in range(K):
            pltpu.make_async_copy(
                bufs[b].at[pl.ds(j * 128, 128)], acc.at[dst_v.at[j]], ssems[b]
            ).wait()

    for q in range(2):
        tq = 2 * cid + q
        pltpu.sync_copy(
            feat_hbm.at[pl.ds(sid * SR, SR), pl.ds(tq * DQ, DQ)],
            table.at[pl.ds(sid * SR, SR)],
        )
        for z in range(ZR // 128):
            pltpu.sync_copy(zero_v, acc.at[pl.ds(sid * ZR + z * 128, 128)])
        plsc.subcore_barrier()

        _fire_gather(0, 0)
        _fire_gather(1, 1)
        for b in (2, 3):
            for j in range(K):
                pltpu.async_copy(
                    zero_v, acc.at[dst_v.at[j]], ssems[b], add=True
                )

        def _round(r, carry):
            ebase = pl.multiple_of(sid * PER_T + r * WB, WB)
            pltpu.sync_copy(w_hbm.at[pl.ds(ebase, WB)], w_v)
            c0 = r * 4
            for b in range(4):
                c = c0 + b
                _drain_gather(b)
                _scale_buf(bufs[b], b * MC)
                nb = (b + 2) % 4
                _drain_scatter(nb)
                if b < 2:
                    _fire_gather(c + 2, nb)
                else:
                    @pl.when(r < NR - 1)
                    def _():
                        _fire_gather(c + 2, nb)
                _fire_scatter(c, b)
            return carry

        lax.fori_loop(0, NR, _round, 0)
        for b in (2, 3):
            _drain_scatter(b)
        plsc.subcore_barrier()

        pltpu.sync_copy(
            acc.at[pl.ds(sid * ZR, ZR)], out_hbm.at[tq, pl.ds(sid * ZR, ZR)]
        )


def _mm_body(p_ref, w_ref, o_ref):
    o_ref[...] = sum(
        jnp.dot(p_ref[k], w_ref[k], preferred_element_type=jnp.float32)
        for k in range(4)
    )


_BM = 400


def _tc_matmul(parts, W4):
    return pl.pallas_call(
        _mm_body,
        grid=(N // _BM,),
        in_specs=[
            pl.BlockSpec((4, _BM, DQ), lambda i: (0, i, 0)),
            pl.BlockSpec((4, DQ, D), lambda i: (0, 0, 0)),
        ],
        out_specs=pl.BlockSpec((_BM, D), lambda i: (i, 0)),
        out_shape=jax.ShapeDtypeStruct((N, D), jnp.float32),
    )(parts, W4)


def kernel(feat, edge_index, edge_weight, W):
    src = edge_index[0]
    dst = edge_index[1]
    pad = EPAD - E
    srcp = jnp.concatenate([src, jnp.zeros((pad,), jnp.int32)]).reshape(
        EPAD // 128, 128
    )
    dstp = jnp.concatenate([dst, jnp.zeros((pad,), jnp.int32)]).reshape(
        EPAD // 128, 128
    )
    wp = jnp.concatenate([edge_weight, jnp.zeros((pad,), jnp.float32)])
    W4 = W.reshape(4, DQ, D)
    parts = _sc_aggregate(feat, srcp, dstp, wp)
    return _tc_matmul(parts, W4)


# speedup vs baseline: 1.0350x; 1.0350x over previous
import functools

import jax
import jax.numpy as jnp
from jax import lax
from jax.experimental import pallas as pl
from jax.experimental.pallas import tpu as pltpu
from jax.experimental.pallas import tpu_sc as plsc

N = 10000
E = 320000
D = 128
DQ = D // 4

NC = 2
NS = 16
L = 16

EPAD = 327680
PER_T = EPAD // NS
ROWS_T = PER_T // 128
MC = 256
NCH = PER_T // MC
NR = NCH // 4
K = MC // 128
WB = 4 * MC
TROWS = NS * 640
ZR = TROWS // NS
SR = N // NS

_mesh = plsc.VectorSubcoreMesh(
    core_axis_name="c", subcore_axis_name="s", num_cores=NC, num_subcores=NS
)


@functools.partial(
    pl.kernel,
    out_type=jax.ShapeDtypeStruct((4, TROWS, DQ), jnp.float32),
    mesh=_mesh,
    scratch_types=[
        pltpu.VMEM((ROWS_T, 128), jnp.int32),
        pltpu.VMEM((ROWS_T, 128), jnp.int32),
        pltpu.VMEM((WB,), jnp.float32),
        pltpu.VMEM((MC, DQ), jnp.float32),
        pltpu.VMEM((MC, DQ), jnp.float32),
        pltpu.VMEM((MC, DQ), jnp.float32),
        pltpu.VMEM((MC, DQ), jnp.float32),
        pltpu.VMEM((128, DQ), jnp.float32),
        pltpu.VMEM_SHARED((TROWS, DQ), jnp.float32),
        pltpu.VMEM_SHARED((TROWS, DQ), jnp.float32),
        pltpu.SemaphoreType.DMA,
        pltpu.SemaphoreType.DMA,
        pltpu.SemaphoreType.DMA,
        pltpu.SemaphoreType.DMA,
        pltpu.SemaphoreType.DMA,
        pltpu.SemaphoreType.DMA,
        pltpu.SemaphoreType.DMA,
        pltpu.SemaphoreType.DMA,
        pltpu.SemaphoreType.DMA,
    ],
    compiler_params=pltpu.CompilerParams(use_tc_tiling_on_sc=False),
)
def _sc_aggregate(feat_hbm, src_hbm, dst_hbm, w_hbm, out_hbm,
                  src_v, dst_v, w_v, buf0, buf1, buf2, buf3, zero_v,
                  table, acc, sg0, sg1, sg2, sg3, ss0, ss1, ss2, ss3, sem_t):
    cid = lax.axis_index("c")
    sid = lax.axis_index("s")
    bufs = (buf0, buf1, buf2, buf3)
    gsems = (sg0, sg1, sg2, sg3)
    ssems = (ss0, ss1, ss2, ss3)

    erow = pl.multiple_of(sid * ROWS_T, 8)
    ecps = [
        pltpu.async_copy(src_hbm.at[pl.ds(erow, ROWS_T)], src_v, sem_t),
        pltpu.async_copy(dst_hbm.at[pl.ds(erow, ROWS_T)], dst_v, sem_t),
    ]

    @plsc.parallel_loop(0, 128, 1, unroll=2)
    def _zero_row(i):
        for j in range(DQ // L):
            zero_v[i, pl.ds(L * j, L)] = jnp.zeros((L,), jnp.float32)

    for cp in ecps:
        cp.wait()

    def _scale_buf(rows, woff):
        @plsc.parallel_loop(0, MC // L, 1, unroll=2)
        def _scale_16rows(t):
            w16 = w_v[pl.ds(woff + t * L, L)]
            for r in range(L):
                i = t * L + r
                wv = jnp.full((L,), w16[r], jnp.float32)
                for j in range(DQ // L):
                    rows[i, pl.ds(L * j, L)] = rows[i, pl.ds(L * j, L)] * wv

    def _fire_gather(c, b):
        for j in range(K):
            pltpu.async_copy(
                table.at[src_v.at[c * K + j]],
                bufs[b].at[pl.ds(j * 128, 128)],
                gsems[b],
            )

    def _drain_gather(b):
        for j in range(K):
            pltpu.make_async_copy(
                table.at[src_v.at[j]], bufs[b].at[pl.ds(j * 128, 128)], gsems[b]
            ).wait()

    def _fire_scatter(c, b):
        for j in range(K):
            pltpu.async_copy(
                bufs[b].at[pl.ds(j * 128, 128)],
                acc.at[dst_v.at[c * K + j]],
                ssems[b],
                add=True,
            )

    def _drain_scatter(b):
        for j in range(K):
            pltpu.make_async_copy(
                bufs[b].at[pl.ds(j * 128, 128)], acc.at[dst_v.at[j]], ssems[b]
            ).wait()

    for q in range(2):
        tq = 2 * cid + q
        pltpu.sync_copy(
            feat_hbm.at[pl.ds(sid * SR, SR), pl.ds(tq * DQ, DQ)],
            table.at[pl.ds(sid * SR, SR)],
        )
        for z in range(ZR // 128):
            pltpu.sync_copy(zero_v, acc.at[pl.ds(sid * ZR + z * 128, 128)])
        plsc.subcore_barrier()

        _fire_gather(0, 0)
        _fire_gather(1, 1)
        for b in (2, 3):
            for j in range(K):
                pltpu.async_copy(
                    zero_v, acc.at[dst_v.at[j]], ssems[b], add=True
                )

        def _round(r, carry):
            ebase = pl.multiple_of(sid * PER_T + r * WB, WB)
            pltpu.sync_copy(w_hbm.at[pl.ds(ebase, WB)], w_v)
            c0 = r * 4
            for b in range(4):
                c = c0 + b
                _drain_gather(b)
                _scale_buf(bufs[b], b * MC)
                nb = (b + 2) % 4
                _drain_scatter(nb)
                if b < 2:
                    _fire_gather(c + 2, nb)
                else:
                    @pl.when(r < NR - 1)
                    def _():
                        _fire_gather(c + 2, nb)
                _fire_scatter(c, b)
            return carry

        lax.fori_loop(0, NR, _round, 0)
        for b in (2, 3):
            _drain_scatter(b)
        plsc.subcore_barrier()

        pltpu.sync_copy(
            acc.at[pl.ds(sid * ZR, ZR)], out_hbm.at[tq, pl.ds(sid * ZR, ZR)]
        )


def _mm_body(p_ref, w_ref, o_ref):
    o_ref[...] = sum(
        jnp.dot(p_ref[k], w_ref[k], preferred_element_type=jnp.float32)
        for k in range(4)
    )


_BM = 400


def _tc_matmul(parts, W4):
    return pl.pallas_call(
        _mm_body,
        grid=(N // _BM,),
        in_specs=[
            pl.BlockSpec((4, _BM, DQ), lambda i: (0, i, 0)),
            pl.BlockSpec((4, DQ, D), lambda i: (0, 0, 0)),
        ],
        out_specs=pl.BlockSpec((_BM, D), lambda i: (i, 0)),
        out_shape=jax.ShapeDtypeStruct((N, D), jnp.float32),
    )(parts, W4)


def kernel(feat, edge_index, edge_weight, W):
    src = edge_index[0]
    dst = edge_index[1]
    pad = EPAD - E
    srcp = jnp.concatenate([src, jnp.zeros((pad,), jnp.int32)]).reshape(
        EPAD // 128, 128
    )
    dstp = jnp.concatenate([dst, jnp.zeros((pad,), jnp.int32)]).reshape(
        EPAD // 128, 128
    )
    wp = jnp.concatenate([edge_weight, jnp.zeros((pad,), jnp.float32)])
    W4 = W.reshape(4, DQ, D)
    parts = _sc_aggregate(feat, srcp, dstp, wp)
    return _tc_matmul(parts, W4)
